# baseline (speedup 1.0000x reference)
import functools

import jax
import jax.numpy as jnp
from jax import lax
from jax.experimental import pallas as pl
from jax.experimental.pallas import tpu as pltpu
from jax.experimental.pallas import tpu_sc as plsc

_NEG_RATIO = 3.0
_EPS = 1e-06

_B, _H, _W = 16, 512, 512
_N = _B * _H * _W

_NC, _NS, _L = 2, 16, 16
_NW = _NC * _NS
_PER_W = _N // _NW
_CHUNK = 16384
_NCHUNK = _PER_W // _CHUNK

_LN2 = 0.69314718
_SQRT2 = 1.4142135381698608


def _poly_log(x):
    xb = lax.bitcast_convert_type(x, jnp.int32)
    e = (xb >> 23) - 127
    man = lax.bitcast_convert_type((xb & 0x007FFFFF) | 0x3F800000, jnp.float32)
    big = man > _SQRT2
    man = jnp.where(big, man * 0.5, man)
    e = e + jnp.where(big, 1, 0)
    r = man - 1.0
    s = r / (2.0 + r)
    t = s * s
    ln1p = s * (2.0 + t * (0.6666667 + t * (0.4 + t * 0.28571430)))
    return e.astype(jnp.float32) * _LN2 + ln1p


_sc_mesh = plsc.VectorSubcoreMesh(core_axis_name="c", subcore_axis_name="s")


_UNROLL = 8

_ROWS = 8192
_ROWS_W = _ROWS // _NW
_CROWS = 32
_NCHUNK2 = _ROWS_W // _CROWS


@functools.partial(
    pl.kernel,
    out_type=jax.ShapeDtypeStruct((_NW, 8, 128), jnp.float32),
    mesh=_sc_mesh,
    compiler_params=pltpu.CompilerParams(use_tc_tiling_on_sc=True),
    scratch_types=[
        pltpu.VMEM((2, _CROWS, 512), jnp.float32),
        pltpu.VMEM((2, _CROWS, 512), jnp.float32),
        pltpu.VMEM((2, _CROWS, 512), jnp.float32),
        pltpu.VMEM((8, 128), jnp.float32),
        pltpu.SemaphoreType.DMA,
        pltpu.SemaphoreType.DMA,
    ],
)
def _sc_reduce(pred_hbm, gt_hbm, mask_hbm, out_hbm, pbuf, gbuf, mbuf, rbuf,
               sem0, sem1):
    wid = lax.axis_index("s") * _NC + lax.axis_index("c")
    base = wid * _ROWS_W
    sems = (sem0, sem1)

    def start(ci, slot):
        sl = pl.ds(base + ci * _CROWS, _CROWS)
        pltpu.async_copy(pred_hbm.at[sl], pbuf.at[slot], sems[slot])
        pltpu.async_copy(gt_hbm.at[sl], gbuf.at[slot], sems[slot])
        pltpu.async_copy(mask_hbm.at[sl], mbuf.at[slot], sems[slot])

    def drain(ci, slot):
        sl = pl.ds(base + ci * _CROWS, _CROWS)
        pltpu.make_async_copy(pred_hbm.at[sl], pbuf.at[slot], sems[slot]).wait()
        pltpu.make_async_copy(gt_hbm.at[sl], gbuf.at[slot], sems[slot]).wait()
        pltpu.make_async_copy(mask_hbm.at[sl], mbuf.at[slot], sems[slot]).wait()

    start(0, 0)
    start(1, 1)

    def pair_body(ci2, carry):
        for slot in (0, 1):
            ci = ci2 * 2 + slot
            drain(ci, slot)

            def row_body(r, carry):
                def vec_body(c, carry):
                    pa, ea, a2, a3 = carry
                    ws, poss, ms = [], [], []
                    for u in range(_UNROLL):
                        sl = pl.ds((c * _UNROLL + u) * _L, _L)
                        p = pbuf[slot, r, sl]
                        g = gbuf[slot, r, sl]
                        m = mbuf[slot, r, sl]
                        xm1 = jnp.where(g > 0.5, p - 1.0, -p)
                        ws.append(m * xm1 + 1.0)
                        poss.append(g * m)
                        ms.append(m)

                    def tree(vals, op):
                        while len(vals) > 1:
                            vals = [op(vals[i], vals[i + 1])
                                    for i in range(0, len(vals), 2)]
                        return vals[0]

                    mul = lambda x_, y_: x_ * y_
                    add = lambda x_, y_: x_ + y_
                    pa = pa * tree(ws, mul)
                    a2 = a2 + tree(poss, add)
                    a3 = a3 + tree(ms, add)
                    pb_ = lax.bitcast_convert_type(pa, jnp.int32)
                    ea = ea + ((pb_ >> 23) - 127)
                    pa = lax.bitcast_convert_type(
                        (pb_ & 0x007FFFFF) | 0x3F800000, jnp.float32)
                    return (pa, ea, a2, a3)

                return lax.fori_loop(0, 512 // (_L * _UNROLL), vec_body, carry)

            carry = plsc.parallel_loop(0, _CROWS, carry=carry)(row_body)

            @pl.when(ci + 2 < _NCHUNK2)
            def _prefetch():
                start(ci + 2, slot)

        return carry

    z = jnp.zeros((_L,), jnp.float32)
    zi = jnp.zeros((_L,), jnp.int32)
    one = jnp.ones((_L,), jnp.float32)
    pa, ea, a2, a3 = lax.fori_loop(
        0, _NCHUNK2 // 2, pair_body, (one, zi, z, z))
    a0 = ea.astype(jnp.float32) * _LN2 + _poly_log(pa)
    rbuf[0, pl.ds(0, _L)] = a0
    rbuf[1, pl.ds(0, _L)] = a2
    rbuf[2, pl.ds(0, _L)] = a3
    pltpu.sync_copy(rbuf, out_hbm.at[wid])


def _fin_body(part_ref, out_ref):
    xx = part_ref[...]
    shape = xx.shape
    q = lax.broadcasted_iota(jnp.int32, shape, 0) % 8
    valid = lax.broadcasted_iota(jnp.int32, shape, 1) < _L
    sel = lambda qq: jnp.sum(jnp.where(jnp.logical_and(q == qq, valid), xx, 0.0))
    s0 = sel(0)
    s1 = sel(1)
    s2 = sel(2)
    pc = jnp.floor(s1)
    ncnt = jnp.floor(s2 - s1)
    kcap = jnp.floor(pc * _NEG_RATIO)
    k = jnp.minimum(ncnt, kcap)
    out_ref[0, 0] = (-s0) / (pc + k + _EPS)
    out_ref[0, 1] = jnp.where(ncnt <= kcap, 1.0, 0.0)


def _finalize_common(part):
    return pl.pallas_call(
        _fin_body,
        out_shape=jax.ShapeDtypeStruct((1, 2), jnp.float32),
        out_specs=pl.BlockSpec(memory_space=pltpu.MemorySpace.SMEM),
    )(part)


_NT = 128
_TMAX = 9.25
_DT = _TMAX / _NT
_RROWS = 256
_RGRID = _N // 1024 // _RROWS


def _rare_body(p_ref, g_ref, m_ref, out_ref, acc, cnt, tsum):
    i = pl.program_id(0)

    @pl.when(i == 0)
    def _init():
        for q in range(4):
            acc[q] = 0.0

        def zbody(j, _):
            cnt[j] = 0.0
            tsum[j] = 0.0
            return 0

        lax.fori_loop(0, _NT + 1, zbody, 0)

    p = p_ref[...]
    g = g_ref[...]
    m = m_ref[...]
    loss = -jnp.log(jnp.where(g > 0.5, p, 1.0 - p))
    pos = g * m
    neg = m - pos
    nl = loss * neg
    acc[0] += jnp.sum(loss * pos)
    acc[1] += jnp.sum(nl)
    acc[2] += jnp.sum(pos)
    acc[3] += jnp.sum(neg)

    def tbody(j, _):
        tj = j.astype(jnp.float32) * _DT
        sel = jnp.logical_and(loss >= tj, neg > 0.5)
        cnt[j] += jnp.sum(jnp.where(sel, 1.0, 0.0))
        tsum[j] += jnp.sum(jnp.where(sel, nl, 0.0))
        return 0

    lax.fori_loop(0, _NT, tbody, 0)

    @pl.when(i == _RGRID - 1)
    def _done():
        pc = jnp.floor(acc[2])
        ncnt = jnp.floor(acc[3])
        k = jnp.minimum(ncnt, jnp.floor(pc * _NEG_RATIO))

        def sbody(j, jstar):
            return jnp.where(cnt[j] >= k, j, jstar)

        jstar = lax.fori_loop(0, _NT, sbody, 0)
        cnt_lo = cnt[jstar]
        sum_lo = tsum[jstar]
        cnt_hi = cnt[jstar + 1]
        sum_hi = tsum[jstar + 1]
        mean_b = (sum_lo - sum_hi) / jnp.maximum(cnt_lo - cnt_hi, 1.0)
        topk = sum_hi + (k - cnt_hi) * mean_b
        out_ref[0, 0] = (acc[0] + topk) / (pc + k + _EPS)


def _rare_topk(pa, ga, ma):
    p2 = pa.reshape(_N // 1024, 1024)
    g2 = ga.reshape(_N // 1024, 1024)
    m2 = ma.reshape(_N // 1024, 1024)
    spec = pl.BlockSpec((_RROWS, 1024), lambda i: (i, 0))
    return pl.pallas_call(
        _rare_body,
        grid=(_RGRID,),
        in_specs=[spec, spec, spec],
        out_specs=pl.BlockSpec(memory_space=pltpu.MemorySpace.SMEM),
        out_shape=jax.ShapeDtypeStruct((1, 1), jnp.float32),
        scratch_shapes=[
            pltpu.SMEM((4,), jnp.float32),
            pltpu.SMEM((_NT + 1,), jnp.float32),
            pltpu.SMEM((_NT + 1,), jnp.float32),
        ],
    )(p2, g2, m2)


def kernel(pred, gt, mask):
    p2 = pred.reshape(_ROWS, 512)
    g2 = gt.reshape(_ROWS, 512)
    m2 = mask.reshape(_ROWS, 512)
    part = _sc_reduce(p2, g2, m2)
    pr = part.reshape(_NW * 8, 128)
    fin = _finalize_common(pr)
    out = lax.cond(
        fin[0, 1] > 0.5,
        lambda ops: ops[0],
        lambda ops: _rare_topk(ops[1], ops[2], ops[3]),
        (fin[:, :1], p2, g2, m2),
    )
    return out.reshape(())


# speedup vs baseline: 1.0010x; 1.0010x over previous
import functools

import jax
import jax.numpy as jnp
from jax import lax
from jax.experimental import pallas as pl
from jax.experimental.pallas import tpu as pltpu
from jax.experimental.pallas import tpu_sc as plsc

_NEG_RATIO = 3.0
_EPS = 1e-06

_B, _H, _W = 16, 512, 512
_N = _B * _H * _W

_NC, _NS, _L = 2, 16, 16
_NW = _NC * _NS
_PER_W = _N // _NW
_CHUNK = 16384
_NCHUNK = _PER_W // _CHUNK

_LN2 = 0.69314718
_SQRT2 = 1.4142135381698608


def _poly_log(x):
    xb = lax.bitcast_convert_type(x, jnp.int32)
    e = (xb >> 23) - 127
    man = lax.bitcast_convert_type((xb & 0x007FFFFF) | 0x3F800000, jnp.float32)
    big = man > _SQRT2
    man = jnp.where(big, man * 0.5, man)
    e = e + jnp.where(big, 1, 0)
    r = man - 1.0
    s = r / (2.0 + r)
    t = s * s
    ln1p = s * (2.0 + t * (0.6666667 + t * (0.4 + t * 0.28571430)))
    return e.astype(jnp.float32) * _LN2 + ln1p


_sc_mesh = plsc.VectorSubcoreMesh(core_axis_name="c", subcore_axis_name="s")


_UNROLL = 8

_ROWS = 8192
_ROWS_W = _ROWS // _NW
_CROWS = 32
_NCHUNK2 = _ROWS_W // _CROWS


@functools.partial(
    pl.kernel,
    out_type=jax.ShapeDtypeStruct((_NW, 8, 128), jnp.float32),
    mesh=_sc_mesh,
    compiler_params=pltpu.CompilerParams(use_tc_tiling_on_sc=True),
    scratch_types=[
        pltpu.VMEM((2, _CROWS, 512), jnp.float32),
        pltpu.VMEM((2, _CROWS, 512), jnp.float32),
        pltpu.VMEM((2, _CROWS, 512), jnp.float32),
        pltpu.VMEM((8, 128), jnp.float32),
        pltpu.SemaphoreType.DMA,
        pltpu.SemaphoreType.DMA,
    ],
)
def _sc_reduce(pred_hbm, gt_hbm, mask_hbm, out_hbm, pbuf, gbuf, mbuf, rbuf,
               sem0, sem1):
    wid = lax.axis_index("s") * _NC + lax.axis_index("c")
    base = wid * _ROWS_W
    sems = (sem0, sem1)

    def start(ci, slot):
        sl = pl.ds(base + ci * _CROWS, _CROWS)
        pltpu.async_copy(pred_hbm.at[sl], pbuf.at[slot], sems[slot])
        pltpu.async_copy(gt_hbm.at[sl], gbuf.at[slot], sems[slot])
        pltpu.async_copy(mask_hbm.at[sl], mbuf.at[slot], sems[slot])

    def drain(ci, slot):
        sl = pl.ds(base + ci * _CROWS, _CROWS)
        pltpu.make_async_copy(pred_hbm.at[sl], pbuf.at[slot], sems[slot]).wait()
        pltpu.make_async_copy(gt_hbm.at[sl], gbuf.at[slot], sems[slot]).wait()
        pltpu.make_async_copy(mask_hbm.at[sl], mbuf.at[slot], sems[slot]).wait()

    start(0, 0)
    start(1, 1)

    def pair_body(ci2, carry):
        for slot in (0, 1):
            ci = ci2 * 2 + slot
            drain(ci, slot)

            def row_body(r, carry):
                def vec_body(c, carry):
                    pa, ea, a2, a3 = carry
                    ws, poss, ms = [], [], []
                    for u in range(_UNROLL):
                        sl = pl.ds((c * _UNROLL + u) * _L, _L)
                        p = pbuf[slot, r, sl]
                        g = gbuf[slot, r, sl]
                        m = mbuf[slot, r, sl]
                        xm1 = jnp.where(g > 0.5, p - 1.0, -p)
                        ws.append(m * xm1 + 1.0)
                        poss.append(g * m)
                        ms.append(m)

                    def tree(vals, op):
                        while len(vals) > 1:
                            vals = [op(vals[i], vals[i + 1])
                                    for i in range(0, len(vals), 2)]
                        return vals[0]

                    mul = lambda x_, y_: x_ * y_
                    add = lambda x_, y_: x_ + y_
                    pa = pa * tree(ws, mul)
                    a2 = a2 + tree(poss, add)
                    a3 = a3 + tree(ms, add)
                    pb_ = lax.bitcast_convert_type(pa, jnp.int32)
                    ea = ea + ((pb_ >> 23) - 127)
                    pa = lax.bitcast_convert_type(
                        (pb_ & 0x007FFFFF) | 0x3F800000, jnp.float32)
                    return (pa, ea, a2, a3)

                return lax.fori_loop(0, 512 // (_L * _UNROLL), vec_body, carry)

            carry = lax.fori_loop(0, _CROWS, row_body, carry)

            @pl.when(ci + 2 < _NCHUNK2)
            def _prefetch():
                start(ci + 2, slot)

        return carry

    z = jnp.zeros((_L,), jnp.float32)
    zi = jnp.zeros((_L,), jnp.int32)
    one = jnp.ones((_L,), jnp.float32)
    pa, ea, a2, a3 = lax.fori_loop(
        0, _NCHUNK2 // 2, pair_body, (one, zi, z, z))
    a0 = ea.astype(jnp.float32) * _LN2 + _poly_log(pa)
    rbuf[0, pl.ds(0, _L)] = a0
    rbuf[1, pl.ds(0, _L)] = a2
    rbuf[2, pl.ds(0, _L)] = a3
    pltpu.sync_copy(rbuf, out_hbm.at[wid])


def _fin_body(part_ref, out_ref):
    xx = part_ref[...]
    shape = xx.shape
    q = lax.broadcasted_iota(jnp.int32, shape, 0) % 8
    valid = lax.broadcasted_iota(jnp.int32, shape, 1) < _L
    sel = lambda qq: jnp.sum(jnp.where(jnp.logical_and(q == qq, valid), xx, 0.0))
    s0 = sel(0)
    s1 = sel(1)
    s2 = sel(2)
    pc = jnp.floor(s1)
    ncnt = jnp.floor(s2 - s1)
    kcap = jnp.floor(pc * _NEG_RATIO)
    k = jnp.minimum(ncnt, kcap)
    out_ref[0, 0] = (-s0) / (pc + k + _EPS)
    out_ref[0, 1] = jnp.where(ncnt <= kcap, 1.0, 0.0)


def _finalize_common(part):
    return pl.pallas_call(
        _fin_body,
        out_shape=jax.ShapeDtypeStruct((1, 2), jnp.float32),
        out_specs=pl.BlockSpec(memory_space=pltpu.MemorySpace.SMEM),
    )(part)


_NT = 128
_TMAX = 9.25
_DT = _TMAX / _NT
_RROWS = 256
_RGRID = _N // 1024 // _RROWS


def _rare_body(p_ref, g_ref, m_ref, out_ref, acc, cnt, tsum):
    i = pl.program_id(0)

    @pl.when(i == 0)
    def _init():
        for q in range(4):
            acc[q] = 0.0

        def zbody(j, _):
            cnt[j] = 0.0
            tsum[j] = 0.0
            return 0

        lax.fori_loop(0, _NT + 1, zbody, 0)

    p = p_ref[...]
    g = g_ref[...]
    m = m_ref[...]
    loss = -jnp.log(jnp.where(g > 0.5, p, 1.0 - p))
    pos = g * m
    neg = m - pos
    nl = loss * neg
    acc[0] += jnp.sum(loss * pos)
    acc[1] += jnp.sum(nl)
    acc[2] += jnp.sum(pos)
    acc[3] += jnp.sum(neg)

    def tbody(j, _):
        tj = j.astype(jnp.float32) * _DT
        sel = jnp.logical_and(loss >= tj, neg > 0.5)
        cnt[j] += jnp.sum(jnp.where(sel, 1.0, 0.0))
        tsum[j] += jnp.sum(jnp.where(sel, nl, 0.0))
        return 0

    lax.fori_loop(0, _NT, tbody, 0)

    @pl.when(i == _RGRID - 1)
    def _done():
        pc = jnp.floor(acc[2])
        ncnt = jnp.floor(acc[3])
        k = jnp.minimum(ncnt, jnp.floor(pc * _NEG_RATIO))

        def sbody(j, jstar):
            return jnp.where(cnt[j] >= k, j, jstar)

        jstar = lax.fori_loop(0, _NT, sbody, 0)
        cnt_lo = cnt[jstar]
        sum_lo = tsum[jstar]
        cnt_hi = cnt[jstar + 1]
        sum_hi = tsum[jstar + 1]
        mean_b = (sum_lo - sum_hi) / jnp.maximum(cnt_lo - cnt_hi, 1.0)
        topk = sum_hi + (k - cnt_hi) * mean_b
        out_ref[0, 0] = (acc[0] + topk) / (pc + k + _EPS)


def _rare_topk(pa, ga, ma):
    p2 = pa.reshape(_N // 1024, 1024)
    g2 = ga.reshape(_N // 1024, 1024)
    m2 = ma.reshape(_N // 1024, 1024)
    spec = pl.BlockSpec((_RROWS, 1024), lambda i: (i, 0))
    return pl.pallas_call(
        _rare_body,
        grid=(_RGRID,),
        in_specs=[spec, spec, spec],
        out_specs=pl.BlockSpec(memory_space=pltpu.MemorySpace.SMEM),
        out_shape=jax.ShapeDtypeStruct((1, 1), jnp.float32),
        scratch_shapes=[
            pltpu.SMEM((4,), jnp.float32),
            pltpu.SMEM((_NT + 1,), jnp.float32),
            pltpu.SMEM((_NT + 1,), jnp.float32),
        ],
    )(p2, g2, m2)


def kernel(pred, gt, mask):
    p2 = pred.reshape(_ROWS, 512)
    g2 = gt.reshape(_ROWS, 512)
    m2 = mask.reshape(_ROWS, 512)
    part = _sc_reduce(p2, g2, m2)
    pr = part.reshape(_NW * 8, 128)
    fin = _finalize_common(pr)
    out = lax.cond(
        fin[0, 1] > 0.5,
        lambda ops: ops[0],
        lambda ops: _rare_topk(ops[1], ops[2], ops[3]),
        (fin[:, :1], p2, g2, m2),
    )
    return out.reshape(())
